# baseline (speedup 1.0000x reference)
import functools

import jax
import jax.numpy as jnp
from jax import lax
from jax.experimental import pallas as pl
from jax.experimental.pallas import tpu as pltpu
from jax.experimental.pallas import tpu_sc as plsc

N = 10000
N_PAD = 10240
E = 320000
P = 20000
F = 128

NC = 2
NS = 16
NW = NC * NS
CHUNK = 128
E_PAD = 327680
ECH = E_PAD // CHUNK
CH_PER_W = ECH // NW
GRP = 4
SUPER = 2 * GRP
NSUPER = CH_PER_W // SUPER
ROWS_PER_TILE = N_PAD // NS
LANES = 16

P_PAD = 20480
POS_PER_W = P_PAD // NW


def _mesh():
    return plsc.VectorSubcoreMesh(core_axis_name="c", subcore_axis_name="s")


def _mm_body(x_ref, w_ref, o_ref):
    o_ref[...] = jnp.dot(x_ref[...], w_ref[...],
                         preferred_element_type=jnp.float32
                         ).astype(o_ref.dtype)


def _mm_partials_body(p_ref, w_ref, o_ref, *, relu):
    h = p_ref[0] + p_ref[1]
    if relu:
        h = jnp.maximum(h, 0.0)
    o_ref[...] = jnp.dot(h, w_ref[...], preferred_element_type=jnp.float32
                         ).astype(o_ref.dtype)


def _tc_mm(x, w, out_dtype=jnp.float32):
    return pl.pallas_call(
        _mm_body,
        out_shape=jax.ShapeDtypeStruct((x.shape[0], w.shape[1]), out_dtype),
    )(x, w)


def _tc_mm_partials(p, w, relu, out_dtype=jnp.float32):
    return pl.pallas_call(
        functools.partial(_mm_partials_body, relu=relu),
        out_shape=jax.ShapeDtypeStruct((p.shape[1], w.shape[1]), out_dtype),
    )(p, w)


def _gcn_aggregate(h, edges2, w2):
    @functools.partial(
        pl.kernel,
        mesh=_mesh(),
        out_type=jax.ShapeDtypeStruct((NC, N_PAD, F), jnp.float32),
        scratch_types=[
            pltpu.VMEM((GRP, 2, CHUNK), jnp.int32),
            pltpu.VMEM((GRP, 2, CHUNK), jnp.int32),
            pltpu.VMEM((GRP, CHUNK), jnp.float32),
            pltpu.VMEM((GRP, CHUNK), jnp.float32),
            pltpu.VMEM((CHUNK, F), jnp.float32),
            pltpu.VMEM((CHUNK, F), jnp.float32),
            pltpu.VMEM_SHARED((N_PAD, F), jnp.float32),
            pltpu.SemaphoreType.DMA,
            pltpu.SemaphoreType.DMA,
            pltpu.SemaphoreType.DMA,
            pltpu.SemaphoreType.DMA,
            pltpu.SemaphoreType.DMA,
            pltpu.SemaphoreType.DMA,
        ],
    )
    def agg(h_hbm, e_hbm, w_hbm, out_hbm,
            e0, e1, wg0, wg1, rows0, fout, acc,
            g0, g1, s0, s1, i0, i1):
        eidx = (e0, e1)
        wg = (wg0, wg1)
        gsem = (g0, g1)
        ssem = (s0, s1)
        isem = (i0, i1)
        cid = lax.axis_index("c")
        sid = lax.axis_index("s")
        wid = sid * NC + cid

        def gwait():
            pltpu.make_async_copy(
                h_hbm.at[e0.at[0, 0]], rows0, g0).wait()

        def swait():
            pltpu.make_async_copy(
                fout, acc.at[e0.at[0, 1]], s0).wait()

        def iload(gp, gidx):
            nb = wbase + gidx * GRP
            pltpu.async_copy(e_hbm.at[pl.ds(nb, GRP)], eidx[gp], isem[gp])
            pltpu.async_copy(w_hbm.at[pl.ds(nb, GRP)], wg[gp], isem[gp])

        def iwait(gp):
            pltpu.make_async_copy(
                e_hbm.at[pl.ds(0, GRP)], eidx[gp], isem[gp]).wait()
            pltpu.make_async_copy(
                w_hbm.at[pl.ds(0, GRP)], wg[gp], isem[gp]).wait()

        def zrow(i, _):
            for j in range(F // LANES):
                fout[i, pl.ds(LANES * j, LANES)] = jnp.zeros(
                    (LANES,), jnp.float32)
            return 0
        lax.fori_loop(0, CHUNK, zrow, 0)
        for m in range(ROWS_PER_TILE // CHUNK):
            pltpu.sync_copy(
                fout,
                acc.at[pl.ds(sid * ROWS_PER_TILE + CHUNK * m, CHUNK)])
        plsc.subcore_barrier()

        wbase = wid * CH_PER_W
        pltpu.sync_copy(e_hbm.at[pl.ds(wbase, GRP)], e0)
        pltpu.sync_copy(w_hbm.at[pl.ds(wbase, GRP)], wg0)
        pltpu.async_copy(h_hbm.at[e0.at[0, 0]], rows0, g0)

        def super_body(t, _):
            for kk in range(SUPER):
                gp = kk // GRP
                c = kk % GRP
                gwait()

                if kk == 1:
                    iload(1, 2 * t + 1)
                if kk == GRP + 1:
                    @pl.when(t < NSUPER - 1)
                    def _ld0():
                        iload(0, 2 * t + 2)

                if kk == 0:
                    @pl.when(t > 0)
                    def _sw():
                        swait()
                else:
                    swait()

                def mul_group(gg, _, gp=gp, c=c):
                    w16 = wg[gp][c, pl.ds(gg * LANES, LANES)]
                    for l in range(LANES):
                        wl = w16[l]
                        i = gg * LANES + l
                        for j in range(F // LANES):
                            sl = pl.ds(LANES * j, LANES)
                            fout[i, sl] = rows0[i, sl] * wl
                    return 0
                lax.fori_loop(0, CHUNK // LANES, mul_group, 0)

                if kk < SUPER - 1:
                    cn = (kk + 1) % GRP
                    gpn = (kk + 1) // GRP
                    if cn == 0:
                        iwait(gpn)
                    pltpu.async_copy(
                        h_hbm.at[eidx[gpn].at[cn, 0]], rows0, g0)
                else:
                    @pl.when(t < NSUPER - 1)
                    def _g0():
                        iwait(0)
                        pltpu.async_copy(h_hbm.at[e0.at[0, 0]], rows0, g0)

                pltpu.async_copy(fout, acc.at[eidx[gp].at[c, 1]],
                                 s0, add=True)
            return 0
        lax.fori_loop(0, NSUPER, super_body, 0)
        swait()
        plsc.subcore_barrier()

        r0 = sid * ROWS_PER_TILE
        pltpu.sync_copy(acc.at[pl.ds(r0, ROWS_PER_TILE)],
                        out_hbm.at[cid, pl.ds(r0, ROWS_PER_TILE)])

    return agg(h, edges2, w2)


POS_CHUNKS_PER_W = POS_PER_W // CHUNK


def _decode(zab, pos0_3d, pos1_3d):
    @functools.partial(
        pl.kernel,
        mesh=_mesh(),
        out_type=jax.ShapeDtypeStruct((P_PAD, LANES), jnp.float32),
        scratch_types=[
            pltpu.VMEM((POS_CHUNKS_PER_W, CHUNK), jnp.int32),
            pltpu.VMEM((POS_CHUNKS_PER_W, CHUNK), jnp.int32),
            pltpu.VMEM((CHUNK, F), jnp.float32),
            pltpu.VMEM((CHUNK, F), jnp.float32),
            pltpu.VMEM((CHUNK, F), jnp.float32),
            pltpu.VMEM((CHUNK, F), jnp.float32),
            pltpu.VMEM((CHUNK, LANES), jnp.float32),
            pltpu.SemaphoreType.DMA,
            pltpu.SemaphoreType.DMA,
            pltpu.SemaphoreType.DMA,
            pltpu.SemaphoreType.DMA,
        ],
    )
    def dec(zab_hbm, p0_hbm, p1_hbm, out_hbm,
            p0_v, p1_v, ra0, ra1, rb0, rb1, o_v, sa0, sa1, sb0, sb1):
        cid = lax.axis_index("c")
        sid = lax.axis_index("s")
        wid = sid * NC + cid
        ra = (ra0, ra1)
        rb = (rb0, rb1)
        sa = (sa0, sa1)
        sb = (sb0, sb1)

        pltpu.sync_copy(p0_hbm.at[wid], p0_v)
        pltpu.sync_copy(p1_hbm.at[wid], p1_v)

        pltpu.async_copy(zab_hbm.at[p0_v.at[0]], ra[0], sa[0])
        pltpu.async_copy(zab_hbm.at[p1_v.at[0]], rb[0], sb[0])
        for k in range(POS_CHUNKS_PER_W):
            b = k % 2
            pltpu.make_async_copy(zab_hbm.at[p0_v.at[k]], ra[b], sa[b]).wait()
            pltpu.make_async_copy(zab_hbm.at[p1_v.at[k]], rb[b], sb[b]).wait()
            if k + 1 < POS_CHUNKS_PER_W:
                pltpu.async_copy(zab_hbm.at[p0_v.at[k + 1]], ra[1 - b],
                                 sa[1 - b])
                pltpu.async_copy(zab_hbm.at[p1_v.at[k + 1]], rb[1 - b],
                                 sb[1 - b])

            def add_body(i, _, b=b):
                o_v[i, pl.ds(0, LANES)] = (ra[b][i, pl.ds(0, LANES)]
                                           + rb[b][i, pl.ds(LANES, LANES)])
                return 0
            lax.fori_loop(0, CHUNK, add_body, 0)
            base = wid * POS_PER_W + k * CHUNK
            pltpu.sync_copy(o_v, out_hbm.at[pl.ds(base, CHUNK)])

    return dec(zab, pos0_3d, pos1_3d)


def kernel(x, edge_index, edge_weight, pos_edge_index, W1, W2, W_lin):
    x = x.astype(jnp.float32)
    src2 = jnp.pad(edge_index[0], (0, E_PAD - E)).reshape(ECH, 1, CHUNK)
    dst2 = jnp.pad(edge_index[1], (0, E_PAD - E)).reshape(ECH, 1, CHUNK)
    edges2 = jnp.concatenate([src2, dst2], axis=1)
    w2 = jnp.pad(edge_weight, (0, E_PAD - E)).reshape(ECH, CHUNK)

    h1 = _tc_mm(x, W1)
    p1 = _gcn_aggregate(h1, edges2, w2)
    h2 = _tc_mm_partials(p1, W2, relu=True)
    p2 = _gcn_aggregate(h2, edges2, w2)

    wcat = jnp.zeros((F, F), jnp.float32)
    wcat = wcat.at[:, 0:2].set(W_lin[:F])
    wcat = wcat.at[:, LANES:LANES + 2].set(W_lin[F:])
    zab = _tc_mm_partials(p2, wcat, relu=False)

    pos = jnp.pad(pos_edge_index, ((0, 0), (0, P_PAD - P)))
    p0_3d = pos[0].reshape(NW, POS_CHUNKS_PER_W, CHUNK)
    p1_3d = pos[1].reshape(NW, POS_CHUNKS_PER_W, CHUNK)
    o2 = _decode(zab, p0_3d, p1_3d)
    return o2[:P, :2]


# speedup vs baseline: 1.1841x; 1.1841x over previous
import functools

import jax
import jax.numpy as jnp
from jax import lax
from jax.experimental import pallas as pl
from jax.experimental.pallas import tpu as pltpu
from jax.experimental.pallas import tpu_sc as plsc

N = 10000
N_PAD = 10240
E = 320000
P = 20000
F = 128

NC = 2
NS = 16
NW = NC * NS
CHUNK = 128
E_PAD = 327680
ECH = E_PAD // CHUNK
CH_PER_W = ECH // NW
GRP = 4
SUPER = 2 * GRP
NSUPER = CH_PER_W // SUPER
ROWS_PER_TILE = N_PAD // NS
LANES = 16

P_PAD = 20480
POS_PER_W = P_PAD // NW


def _mesh():
    return plsc.VectorSubcoreMesh(core_axis_name="c", subcore_axis_name="s")


def _mm_body(x_ref, w_ref, o_ref):
    o_ref[...] = jnp.dot(x_ref[...], w_ref[...],
                         preferred_element_type=jnp.float32)


def _mm_partials_body(p_ref, w_ref, o_ref, *, relu):
    h = p_ref[0] + p_ref[1]
    if relu:
        h = jnp.maximum(h, 0.0)
    o_ref[...] = jnp.dot(h, w_ref[...], preferred_element_type=jnp.float32)


def _tc_mm(x, w):
    return pl.pallas_call(
        _mm_body,
        out_shape=jax.ShapeDtypeStruct((x.shape[0], w.shape[1]), jnp.float32),
    )(x, w)


def _tc_mm_partials(p, w, relu):
    return pl.pallas_call(
        functools.partial(_mm_partials_body, relu=relu),
        out_shape=jax.ShapeDtypeStruct((p.shape[1], w.shape[1]), jnp.float32),
    )(p, w)


def _gcn_aggregate(h, edges2, w2):
    @functools.partial(
        pl.kernel,
        mesh=_mesh(),
        out_type=jax.ShapeDtypeStruct((NC, N_PAD, F), jnp.float32),
        scratch_types=[
            pltpu.VMEM((GRP, 2, CHUNK), jnp.int32),
            pltpu.VMEM((GRP, 2, CHUNK), jnp.int32),
            pltpu.VMEM((GRP, CHUNK), jnp.float32),
            pltpu.VMEM((GRP, CHUNK), jnp.float32),
            pltpu.VMEM((CHUNK, F), jnp.float32),
            pltpu.VMEM((CHUNK, F), jnp.float32),
            pltpu.VMEM_SHARED((N_PAD, F), jnp.float32),
            pltpu.SemaphoreType.DMA,
            pltpu.SemaphoreType.DMA,
            pltpu.SemaphoreType.DMA,
            pltpu.SemaphoreType.DMA,
            pltpu.SemaphoreType.DMA,
            pltpu.SemaphoreType.DMA,
        ],
    )
    def agg(h_hbm, e_hbm, w_hbm, out_hbm,
            e0, e1, wg0, wg1, rows0, rows1, acc,
            g0, g1, s0, s1, i0, i1):
        rows = (rows0, rows1)
        eidx = (e0, e1)
        wg = (wg0, wg1)
        gsem = (g0, g1)
        ssem = (s0, s1)
        isem = (i0, i1)
        cid = lax.axis_index("c")
        sid = lax.axis_index("s")
        wid = sid * NC + cid

        def gwait(b):
            pltpu.make_async_copy(
                h_hbm.at[e0.at[0, 0]], rows[b], gsem[b]).wait()

        def swait(b):
            pltpu.make_async_copy(
                rows[b], acc.at[e0.at[0, 1]], ssem[b]).wait()

        def iload(gp, gidx):
            nb = wbase + gidx * GRP
            pltpu.async_copy(e_hbm.at[pl.ds(nb, GRP)], eidx[gp], isem[gp])
            pltpu.async_copy(w_hbm.at[pl.ds(nb, GRP)], wg[gp], isem[gp])

        def iwait(gp):
            pltpu.make_async_copy(
                e_hbm.at[pl.ds(0, GRP)], eidx[gp], isem[gp]).wait()
            pltpu.make_async_copy(
                w_hbm.at[pl.ds(0, GRP)], wg[gp], isem[gp]).wait()

        def zrow(i, _):
            for j in range(F // LANES):
                rows0[i, pl.ds(LANES * j, LANES)] = jnp.zeros(
                    (LANES,), jnp.float32)
            return 0
        lax.fori_loop(0, CHUNK, zrow, 0)
        for m in range(ROWS_PER_TILE // CHUNK):
            pltpu.sync_copy(
                rows0,
                acc.at[pl.ds(sid * ROWS_PER_TILE + CHUNK * m, CHUNK)])
        plsc.subcore_barrier()

        wbase = wid * CH_PER_W
        pltpu.sync_copy(e_hbm.at[pl.ds(wbase, GRP)], e0)
        pltpu.sync_copy(w_hbm.at[pl.ds(wbase, GRP)], wg0)
        pltpu.async_copy(h_hbm.at[e0.at[0, 0]], rows0, g0)

        def super_body(t, _):
            for kk in range(SUPER):
                b = kk % 2
                gp = kk // GRP
                c = kk % GRP
                gwait(b)

                if kk == 1:
                    iload(1, 2 * t + 1)
                if kk == GRP + 1:
                    @pl.when(t < NSUPER - 1)
                    def _ld0():
                        iload(0, 2 * t + 2)

                if kk == 0:
                    pltpu.async_copy(
                        h_hbm.at[e0.at[1, 0]], rows1, g1)
                elif kk < SUPER - 1:
                    cn = (kk + 1) % GRP
                    gpn = (kk + 1) // GRP
                    if cn == 0:
                        iwait(gpn)
                    pltpu.async_copy(
                        h_hbm.at[eidx[gpn].at[cn, 0]], rows[1 - b],
                        gsem[1 - b])
                else:
                    @pl.when(t < NSUPER - 1)
                    def _g0():
                        iwait(0)
                        pltpu.async_copy(
                            h_hbm.at[e0.at[0, 0]], rows[1 - b],
                            gsem[1 - b])

                def mul_group(gg, _, gp=gp, c=c, b=b):
                    w16 = wg[gp][c, pl.ds(gg * LANES, LANES)]
                    for l in range(LANES):
                        wl = w16[l]
                        i = gg * LANES + l
                        for j in range(F // LANES):
                            sl = pl.ds(LANES * j, LANES)
                            rows[b][i, sl] = rows[b][i, sl] * wl
                    return 0
                lax.fori_loop(0, CHUNK // LANES, mul_group, 0)

                pltpu.sync_copy(rows[b], acc.at[eidx[gp].at[c, 1]],
                                add=True)
            return 0
        lax.fori_loop(0, NSUPER, super_body, 0)
        plsc.subcore_barrier()

        r0 = sid * ROWS_PER_TILE
        pltpu.sync_copy(acc.at[pl.ds(r0, ROWS_PER_TILE)],
                        out_hbm.at[cid, pl.ds(r0, ROWS_PER_TILE)])

    return agg(h, edges2, w2)


POS_CHUNKS_PER_W = POS_PER_W // CHUNK


def _decode(zab, pos0_3d, pos1_3d):
    @functools.partial(
        pl.kernel,
        mesh=_mesh(),
        out_type=jax.ShapeDtypeStruct((P_PAD, LANES), jnp.float32),
        scratch_types=[
            pltpu.VMEM((POS_CHUNKS_PER_W, CHUNK), jnp.int32),
            pltpu.VMEM((POS_CHUNKS_PER_W, CHUNK), jnp.int32),
            pltpu.VMEM((CHUNK, F), jnp.float32),
            pltpu.VMEM((CHUNK, F), jnp.float32),
            pltpu.VMEM((CHUNK, F), jnp.float32),
            pltpu.VMEM((CHUNK, F), jnp.float32),
            pltpu.VMEM((CHUNK, LANES), jnp.float32),
            pltpu.SemaphoreType.DMA,
            pltpu.SemaphoreType.DMA,
            pltpu.SemaphoreType.DMA,
            pltpu.SemaphoreType.DMA,
        ],
    )
    def dec(zab_hbm, p0_hbm, p1_hbm, out_hbm,
            p0_v, p1_v, ra0, ra1, rb0, rb1, o_v, sa0, sa1, sb0, sb1):
        cid = lax.axis_index("c")
        sid = lax.axis_index("s")
        wid = sid * NC + cid
        ra = (ra0, ra1)
        rb = (rb0, rb1)
        sa = (sa0, sa1)
        sb = (sb0, sb1)

        pltpu.sync_copy(p0_hbm.at[wid], p0_v)
        pltpu.sync_copy(p1_hbm.at[wid], p1_v)

        pltpu.async_copy(zab_hbm.at[p0_v.at[0]], ra[0], sa[0])
        pltpu.async_copy(zab_hbm.at[p1_v.at[0]], rb[0], sb[0])
        for k in range(POS_CHUNKS_PER_W):
            b = k % 2
            pltpu.make_async_copy(zab_hbm.at[p0_v.at[k]], ra[b], sa[b]).wait()
            pltpu.make_async_copy(zab_hbm.at[p1_v.at[k]], rb[b], sb[b]).wait()
            if k + 1 < POS_CHUNKS_PER_W:
                pltpu.async_copy(zab_hbm.at[p0_v.at[k + 1]], ra[1 - b],
                                 sa[1 - b])
                pltpu.async_copy(zab_hbm.at[p1_v.at[k + 1]], rb[1 - b],
                                 sb[1 - b])

            def add_body(i, _, b=b):
                o_v[i, pl.ds(0, LANES)] = (ra[b][i, pl.ds(0, LANES)]
                                           + rb[b][i, pl.ds(LANES, LANES)])
                return 0
            lax.fori_loop(0, CHUNK, add_body, 0)
            base = wid * POS_PER_W + k * CHUNK
            pltpu.sync_copy(o_v, out_hbm.at[pl.ds(base, CHUNK)])

    return dec(zab, pos0_3d, pos1_3d)


def kernel(x, edge_index, edge_weight, pos_edge_index, W1, W2, W_lin):
    x = x.astype(jnp.float32)
    src2 = jnp.pad(edge_index[0], (0, E_PAD - E)).reshape(ECH, 1, CHUNK)
    dst2 = jnp.pad(edge_index[1], (0, E_PAD - E)).reshape(ECH, 1, CHUNK)
    edges2 = jnp.concatenate([src2, dst2], axis=1)
    w2 = jnp.pad(edge_weight, (0, E_PAD - E)).reshape(ECH, CHUNK)

    h1 = _tc_mm(x, W1)
    p1 = _gcn_aggregate(h1, edges2, w2)
    h2 = _tc_mm_partials(p1, W2, relu=True)
    p2 = _gcn_aggregate(h2, edges2, w2)

    wcat = jnp.zeros((F, F), jnp.float32)
    wcat = wcat.at[:, 0:2].set(W_lin[:F])
    wcat = wcat.at[:, LANES:LANES + 2].set(W_lin[F:])
    zab = _tc_mm_partials(p2, wcat, relu=False)

    pos = jnp.pad(pos_edge_index, ((0, 0), (0, P_PAD - P)))
    p0_3d = pos[0].reshape(NW, POS_CHUNKS_PER_W, CHUNK)
    p1_3d = pos[1].reshape(NW, POS_CHUNKS_PER_W, CHUNK)
    o2 = _decode(zab, p0_3d, p1_3d)
    return o2[:P, :2]
